# Initial kernel scaffold; baseline (speedup 1.0000x reference)
import jax
import jax.numpy as jnp
from jax.experimental import pallas as pl


def kernel(x, edge_index, edge_weight, W, b):
    raise NotImplementedError("write your pallas kernel here")


# speedup vs baseline: 4.4103x; 4.4103x over previous
import jax
import jax.numpy as jnp
from jax import lax
from jax.experimental import pallas as pl
from jax.experimental.pallas import tpu as pltpu
from jax.experimental.pallas import tpu_sc as plsc

_NC = 2
_NS = 16
_NW = _NC * _NS
_C = 128
_L = 16


def _sc_spmm(x, src, dst, w, zeros):
    n, d = x.shape
    npad = zeros.shape[0]
    ep = src.shape[0]
    e_per_w = ep // _NW
    nch = e_per_w // _C
    rows_per_tile = npad // _NS
    nvec = d // _L
    mesh = plsc.VectorSubcoreMesh(core_axis_name="c", subcore_axis_name="s",
                                  num_cores=_NC, num_subcores=_NS)

    def body(x_hbm, src_hbm, dst_hbm, w_hbm, z_hbm, out_hbm,
             acc, srcv0, srcv1, dstv0, dstv1, wv0, wv1, rows0, rows1,
             sem0, sem1):
        cid = lax.axis_index("c")
        sid = lax.axis_index("s")
        wid = sid * _NC + cid
        base = wid * e_per_w
        srcv = (srcv0, srcv1)
        dstv = (dstv0, dstv1)
        wv = (wv0, wv1)
        rows = (rows0, rows1)
        sem = (sem0, sem1)

        r0 = sid * rows_per_tile
        pltpu.sync_copy(z_hbm.at[pl.ds(r0, rows_per_tile)],
                        acc.at[pl.ds(r0, rows_per_tile)])
        plsc.subcore_barrier()

        def fetch(jj, b):
            off = base + jj * _C
            pltpu.sync_copy(src_hbm.at[pl.ds(off, _C)], srcv[b])
            pltpu.sync_copy(dst_hbm.at[pl.ds(off, _C)], dstv[b])
            pltpu.sync_copy(w_hbm.at[pl.ds(off, _C)], wv[b])
            pltpu.async_copy(x_hbm.at[srcv[b]], rows[b], sem[b])

        fetch(0, 0)
        fetch(1, 1)

        def chunk_body(j, carry):
            for b in range(2):
                jj = 2 * j + b
                pltpu.make_async_copy(x_hbm.at[srcv[b]], rows[b],
                                      sem[b]).wait()

                def scale(g, c):
                    wg = wv[b][pl.ds(g * _L, _L)]
                    for l in range(_L):
                        wvec = jnp.full((_L,), wg[l], dtype=jnp.float32)
                        e = g * _L + l
                        for k in range(nvec):
                            sl = pl.ds(k * _L, _L)
                            rows[b][e, sl] = rows[b][e, sl] * wvec
                    return c

                lax.fori_loop(0, _C // _L, scale, 0)

                pltpu.sync_copy(rows[b], acc.at[dstv[b]], add=True)

                @pl.when(jj + 2 < nch)
                def _():
                    fetch(jj + 2, b)
            return carry

        lax.fori_loop(0, nch // 2, chunk_body, 0)

        plsc.subcore_barrier()
        pltpu.sync_copy(acc.at[pl.ds(r0, rows_per_tile)],
                        out_hbm.at[cid, pl.ds(r0, rows_per_tile)])

    return pl.kernel(
        body,
        out_type=jax.ShapeDtypeStruct((_NC, npad, d), jnp.float32),
        mesh=mesh,
        scratch_types=[
            pltpu.VMEM_SHARED((npad, d), jnp.float32),
            pltpu.VMEM((_C,), jnp.int32),
            pltpu.VMEM((_C,), jnp.int32),
            pltpu.VMEM((_C,), jnp.int32),
            pltpu.VMEM((_C,), jnp.int32),
            pltpu.VMEM((_C,), jnp.float32),
            pltpu.VMEM((_C,), jnp.float32),
            pltpu.VMEM((_C, d), jnp.float32),
            pltpu.VMEM((_C, d), jnp.float32),
            pltpu.SemaphoreType.DMA,
            pltpu.SemaphoreType.DMA,
        ],
    )(x, src, dst, w, zeros)


def _tc_linear(partial, W, b2, n):
    d = partial.shape[2]
    blk = 1000

    def body(p_ref, w_ref, b_ref, o_ref):
        h = p_ref[0] + p_ref[1]
        o_ref[...] = lax.dot_general(
            h, w_ref[...], (((1,), (1,)), ((), ())),
            preferred_element_type=jnp.float32) + b_ref[...]

    return pl.pallas_call(
        body,
        grid=(n // blk,),
        in_specs=[
            pl.BlockSpec((2, blk, d), lambda i: (0, i, 0)),
            pl.BlockSpec((d, d), lambda i: (0, 0)),
            pl.BlockSpec((1, d), lambda i: (0, 0)),
        ],
        out_specs=pl.BlockSpec((blk, d), lambda i: (i, 0)),
        out_shape=jax.ShapeDtypeStruct((n, d), jnp.float32),
    )(partial, W, b2)


def kernel(x, edge_index, edge_weight, W, b):
    n, d = x.shape
    e = edge_index.shape[1]
    quantum = _NW * _C * 2
    ep = quantum * ((e + quantum - 1) // quantum)
    pad = ep - e
    src = jnp.concatenate(
        [edge_index[0].astype(jnp.int32), jnp.zeros((pad,), jnp.int32)])
    dst = jnp.concatenate(
        [edge_index[1].astype(jnp.int32), jnp.zeros((pad,), jnp.int32)])
    w = jnp.concatenate(
        [edge_weight.astype(jnp.float32), jnp.zeros((pad,), jnp.float32)])
    nq = 8 * _NS
    npad = nq * ((n + nq - 1) // nq)
    zeros = jnp.zeros((npad, d), jnp.float32)
    partial = _sc_spmm(x, src, dst, w, zeros)
    return _tc_linear(partial, W, b.reshape(1, d), n)


# speedup vs baseline: 6.6844x; 1.5156x over previous
import jax
import jax.numpy as jnp
from jax import lax
from jax.experimental import pallas as pl
from jax.experimental.pallas import tpu as pltpu
from jax.experimental.pallas import tpu_sc as plsc

_NC = 2
_NS = 16
_NW = _NC * _NS
_C = 112
_L = 16
_NBUF = 3


def _sc_spmm(x, edata, wdata, zeros):
    n, d = x.shape
    npad = zeros.shape[0]
    nch = edata.shape[1]
    rows_per_tile = npad // _NS
    nvec = d // _L
    mesh = plsc.VectorSubcoreMesh(core_axis_name="c", subcore_axis_name="s",
                                  num_cores=_NC, num_subcores=_NS)

    def body(x_hbm, e_hbm, w_hbm, z_hbm, out_hbm,
             acc, ebuf, wbuf, rows0, rows1, rows2,
             gsem0, gsem1, gsem2, ssem0, ssem1, ssem2):
        cid = lax.axis_index("c")
        sid = lax.axis_index("s")
        wid = sid * _NC + cid
        rows = (rows0, rows1, rows2)
        gsem = (gsem0, gsem1, gsem2)
        ssem = (ssem0, ssem1, ssem2)

        r0 = sid * rows_per_tile
        pltpu.sync_copy(z_hbm.at[pl.ds(r0, rows_per_tile)],
                        acc.at[pl.ds(r0, rows_per_tile)])
        plsc.subcore_barrier()

        def fetch_and_gather(jj, s):
            pltpu.sync_copy(e_hbm.at[wid, jj], ebuf.at[s])
            pltpu.sync_copy(w_hbm.at[wid, jj], wbuf.at[s])
            pltpu.async_copy(x_hbm.at[ebuf.at[s, 0]], rows[s], gsem[s])

        fetch_and_gather(0, 0)
        fetch_and_gather(1, 1)

        def triple(t, carry):
            for b in range(_NBUF):
                jj = _NBUF * t + b
                sn = (b + 2) % _NBUF

                pltpu.make_async_copy(
                    x_hbm.at[ebuf.at[b, 0]], rows[b], gsem[b]).wait()

                @plsc.parallel_loop(0, _C // _L)
                def _(g):
                    wg = wbuf[b, pl.ds(g * _L, _L)]
                    for l in range(_L):
                        wvec = jnp.full((_L,), wg[l], dtype=jnp.float32)
                        e = g * _L + l
                        for k in range(nvec):
                            sl = pl.ds(k * _L, _L)
                            rows[b][e, sl] = rows[b][e, sl] * wvec

                @pl.when(jj >= 1)
                def _():
                    pltpu.make_async_copy(
                        rows[sn], acc.at[ebuf.at[sn, 1]], ssem[sn]).wait()

                @pl.when(jj + 2 < nch)
                def _():
                    fetch_and_gather(jj + 2, sn)

                pltpu.make_async_copy(
                    rows[b], acc.at[ebuf.at[b, 1]], ssem[b]).start(add=True)
            return carry

        lax.fori_loop(0, nch // _NBUF, triple, 0)

        sl_ = (nch - 1) % _NBUF
        pltpu.make_async_copy(
            rows[sl_], acc.at[ebuf.at[sl_, 1]], ssem[sl_]).wait()

        plsc.subcore_barrier()
        pltpu.sync_copy(acc.at[pl.ds(r0, rows_per_tile)],
                        out_hbm.at[cid, pl.ds(r0, rows_per_tile)])

    return pl.kernel(
        body,
        out_type=jax.ShapeDtypeStruct((_NC, npad, d), jnp.float32),
        mesh=mesh,
        scratch_types=[
            pltpu.VMEM_SHARED((npad, d), jnp.float32),
            pltpu.VMEM((_NBUF, 2, _C), jnp.int32),
            pltpu.VMEM((_NBUF, _C), jnp.float32),
            pltpu.VMEM((_C, d), jnp.float32),
            pltpu.VMEM((_C, d), jnp.float32),
            pltpu.VMEM((_C, d), jnp.float32),
            pltpu.SemaphoreType.DMA,
            pltpu.SemaphoreType.DMA,
            pltpu.SemaphoreType.DMA,
            pltpu.SemaphoreType.DMA,
            pltpu.SemaphoreType.DMA,
            pltpu.SemaphoreType.DMA,
        ],
    )(x, edata, wdata, zeros)


def _tc_linear(partial, W, b2, n):
    d = partial.shape[2]
    blk = 1000

    def body(p_ref, w_ref, b_ref, o_ref):
        h = p_ref[0] + p_ref[1]
        o_ref[...] = lax.dot_general(
            h, w_ref[...], (((1,), (1,)), ((), ())),
            preferred_element_type=jnp.float32) + b_ref[...]

    return pl.pallas_call(
        body,
        grid=(n // blk,),
        in_specs=[
            pl.BlockSpec((2, blk, d), lambda i: (0, i, 0)),
            pl.BlockSpec((d, d), lambda i: (0, 0)),
            pl.BlockSpec((1, d), lambda i: (0, 0)),
        ],
        out_specs=pl.BlockSpec((blk, d), lambda i: (i, 0)),
        out_shape=jax.ShapeDtypeStruct((n, d), jnp.float32),
    )(partial, W, b2)


def kernel(x, edge_index, edge_weight, W, b):
    n, d = x.shape
    e = edge_index.shape[1]
    quantum = _NW * _C * _NBUF
    ep = quantum * ((e + quantum - 1) // quantum)
    pad = ep - e
    nch = ep // (_NW * _C)
    src = jnp.concatenate(
        [edge_index[0].astype(jnp.int32), jnp.zeros((pad,), jnp.int32)])
    dst = jnp.concatenate(
        [edge_index[1].astype(jnp.int32), jnp.zeros((pad,), jnp.int32)])
    w = jnp.concatenate(
        [edge_weight.astype(jnp.float32), jnp.zeros((pad,), jnp.float32)])
    edata = jnp.stack([src.reshape(_NW, nch, _C),
                       dst.reshape(_NW, nch, _C)], axis=2)
    wdata = w.reshape(_NW, nch, _C)
    nq = 8 * _NS
    npad = nq * ((n + nq - 1) // nq)
    zeros = jnp.zeros((npad, d), jnp.float32)
    partial = _sc_spmm(x, edata, wdata, zeros)
    return _tc_linear(partial, W, b.reshape(1, d), n)
